# baseline (speedup 1.0000x reference)
import functools

import jax
import jax.numpy as jnp
from jax import lax
from jax.experimental import pallas as pl
from jax.experimental.pallas import tpu as pltpu
from jax.experimental.pallas import tpu_sc as plsc

L = 16
RU = 8


def _make_sc_kernel(B, C, J, rows_per_w, rchunk):
  nchunks = rows_per_w // rchunk
  xsz = rchunk * C
  mesh = plsc.VectorSubcoreMesh(core_axis_name="c", subcore_axis_name="s")

  @functools.partial(
      pl.kernel,
      mesh=mesh,
      out_type=jax.ShapeDtypeStruct((B, J), jnp.float32),
      scratch_types=[
          pltpu.VMEM((J,), jnp.int32),
          pltpu.VMEM((2, rchunk // 2, 2 * C), jnp.float32),
          pltpu.VMEM((2, rchunk, J), jnp.float32),
          pltpu.SemaphoreType.DMA((2,)),
          pltpu.SemaphoreType.DMA((2,)),
      ],
      compiler_params=pltpu.CompilerParams(needs_layout_passes=False),
  )
  def k(x_hbm, idx_hbm, out_hbm, idx_v, xin_v, oout_v, sin, sout):
    wid = lax.axis_index("s") * 2 + lax.axis_index("c")
    base = wid * rows_per_w
    base2 = wid * (rows_per_w // 2)

    pltpu.sync_copy(idx_hbm, idx_v)
    pltpu.async_copy(x_hbm.at[pl.ds(base2, rchunk // 2)], xin_v.at[0], sin.at[0])

    def chunk_body(ci, _):
      p = lax.rem(ci, 2)
      r0 = base + ci * rchunk
      r02 = base2 + ci * (rchunk // 2)

      @pl.when(ci + 1 < nchunks)
      def _prefetch():
        pltpu.async_copy(x_hbm.at[pl.ds(r02 + rchunk // 2, rchunk // 2)],
                         xin_v.at[1 - p], sin.at[1 - p])

      pltpu.make_async_copy(x_hbm.at[pl.ds(r02, rchunk // 2)],
                            xin_v.at[p], sin.at[p]).wait()

      @pl.when(ci >= 2)
      def _drain():
        pltpu.make_async_copy(
            oout_v.at[p],
            out_hbm.at[pl.ds(r0 - 2 * rchunk, rchunk)],
            sout.at[p]).wait()

      ob = oout_v.at[p]
      xb = xin_v.at[p]
      for j in range(J // L):
        iv = idx_v[pl.ds(j * L, L)]
        iv1 = iv + jnp.full((L,), C, jnp.int32)
        ivs = (iv, iv1)

        @plsc.parallel_loop(0, rchunk, RU)
        def _rows(r):
          rsp = [jnp.broadcast_to((r + 2 * kk2) // 2, (L,)).astype(jnp.int32)
                 for kk2 in range(RU // 2)]
          vals = [plsc.load_gather(xb, [rsp[kk // 2], ivs[kk % 2]])
                  for kk in range(RU)]
          for kk in range(RU):
            ob[r + kk, pl.ds(j * L, L)] = vals[kk]

      pltpu.async_copy(ob, out_hbm.at[pl.ds(r0, rchunk)], sout.at[p])

    lax.fori_loop(0, nchunks, chunk_body, None)

    end0 = base + (nchunks - 2) * rchunk
    pltpu.make_async_copy(oout_v.at[0], out_hbm.at[pl.ds(end0, rchunk)],
                          sout.at[0]).wait()
    pltpu.make_async_copy(oout_v.at[1],
                          out_hbm.at[pl.ds(end0 + rchunk, rchunk)],
                          sout.at[1]).wait()

  return k


def kernel(x, hierarchy_mapping_idx):
  B, C = x.shape
  J = hierarchy_mapping_idx.shape[0]
  rows_per_w = B // 32
  rchunk = 32
  k = _make_sc_kernel(B, C, J, rows_per_w, rchunk)
  return k(x.reshape(B // 2, 2 * C), hierarchy_mapping_idx.astype(jnp.int32))


# speedup vs baseline: 1.4321x; 1.4321x over previous
import functools

import jax
import jax.numpy as jnp
from jax import lax
from jax.experimental import pallas as pl
from jax.experimental.pallas import tpu as pltpu
from jax.experimental.pallas import tpu_sc as plsc

L = 16
RU = 8


def _make_sc_kernel(B, C, J, rows_per_w, rchunk):
  nchunks = rows_per_w // rchunk
  xsz = rchunk * C
  mesh = plsc.VectorSubcoreMesh(core_axis_name="c", subcore_axis_name="s")

  @functools.partial(
      pl.kernel,
      mesh=mesh,
      out_type=jax.ShapeDtypeStruct((B, J), jnp.float32),
      scratch_types=[
          pltpu.VMEM((J,), jnp.int32),
          pltpu.VMEM((2, rchunk, C), jnp.float32),
          pltpu.VMEM((2, rchunk, J), jnp.float32),
          pltpu.SemaphoreType.DMA((2,)),
          pltpu.SemaphoreType.DMA((2,)),
      ],
      compiler_params=pltpu.CompilerParams(needs_layout_passes=False),
  )
  def k(x_hbm, idx_hbm, out_hbm, idx_v, xin_v, oout_v, sin, sout):
    wid = lax.axis_index("s") * 2 + lax.axis_index("c")
    base = wid * rows_per_w

    pltpu.sync_copy(idx_hbm, idx_v)
    pltpu.async_copy(x_hbm.at[pl.ds(base, rchunk)], xin_v.at[0], sin.at[0])

    def chunk_body(ci, _):
      p = lax.rem(ci, 2)
      r0 = base + ci * rchunk

      @pl.when(ci + 1 < nchunks)
      def _prefetch():
        pltpu.async_copy(x_hbm.at[pl.ds(r0 + rchunk, rchunk)],
                         xin_v.at[1 - p], sin.at[1 - p])

      pltpu.make_async_copy(x_hbm.at[pl.ds(r0, rchunk)],
                            xin_v.at[p], sin.at[p]).wait()

      @pl.when(ci >= 2)
      def _drain():
        pltpu.make_async_copy(
            oout_v.at[p],
            out_hbm.at[pl.ds(r0 - 2 * rchunk, rchunk)],
            sout.at[p]).wait()

      ob = oout_v.at[p]
      xb = xin_v.at[p]
      for j in range(J // L):
        iv = idx_v[pl.ds(j * L, L)]

        @plsc.parallel_loop(0, rchunk, RU)
        def _rows(r):
          vals = [plsc.load_gather(xb.at[r + kk], [iv])
                  for kk in range(RU)]
          for kk in range(RU):
            ob[r + kk, pl.ds(j * L, L)] = vals[kk]

      pltpu.async_copy(ob, out_hbm.at[pl.ds(r0, rchunk)], sout.at[p])

    lax.fori_loop(0, nchunks, chunk_body, None)

    end0 = base + (nchunks - 2) * rchunk
    pltpu.make_async_copy(oout_v.at[0], out_hbm.at[pl.ds(end0, rchunk)],
                          sout.at[0]).wait()
    pltpu.make_async_copy(oout_v.at[1],
                          out_hbm.at[pl.ds(end0 + rchunk, rchunk)],
                          sout.at[1]).wait()

  return k


def kernel(x, hierarchy_mapping_idx):
  B, C = x.shape
  J = hierarchy_mapping_idx.shape[0]
  rows_per_w = B // 32
  rchunk = 32
  k = _make_sc_kernel(B, C, J, rows_per_w, rchunk)
  return k(x, hierarchy_mapping_idx.astype(jnp.int32))


# speedup vs baseline: 1.4758x; 1.0305x over previous
import functools

import jax
import jax.numpy as jnp
from jax import lax
from jax.experimental import pallas as pl
from jax.experimental.pallas import tpu as pltpu
from jax.experimental.pallas import tpu_sc as plsc

L = 16
RU = 8


def _make_sc_kernel(B, C, J, rows_per_w, rchunk):
  nchunks = rows_per_w // rchunk
  xsz = rchunk * C
  mesh = plsc.VectorSubcoreMesh(core_axis_name="c", subcore_axis_name="s")

  @functools.partial(
      pl.kernel,
      mesh=mesh,
      out_type=jax.ShapeDtypeStruct((B, J), jnp.float32),
      scratch_types=[
          pltpu.VMEM((J,), jnp.int32),
          pltpu.VMEM((2, rchunk, C), jnp.float32),
          pltpu.VMEM((2, rchunk, J), jnp.float32),
          pltpu.SemaphoreType.DMA((2,)),
          pltpu.SemaphoreType.DMA((2,)),
      ],
      compiler_params=pltpu.CompilerParams(needs_layout_passes=False),
  )
  def k(x_hbm, idx_hbm, out_hbm, idx_v, xin_v, oout_v, sin, sout):
    wid = lax.axis_index("s") * 2 + lax.axis_index("c")
    base = wid * rows_per_w

    pltpu.sync_copy(idx_hbm, idx_v)
    pltpu.async_copy(x_hbm.at[pl.ds(base, rchunk)], xin_v.at[0], sin.at[0])

    def chunk_body(ci, _):
      p = lax.rem(ci, 2)
      r0 = base + ci * rchunk

      @pl.when(ci + 1 < nchunks)
      def _prefetch():
        pltpu.async_copy(x_hbm.at[pl.ds(r0 + rchunk, rchunk)],
                         xin_v.at[1 - p], sin.at[1 - p])

      pltpu.make_async_copy(x_hbm.at[pl.ds(r0, rchunk)],
                            xin_v.at[p], sin.at[p]).wait()

      @pl.when(ci >= 2)
      def _drain():
        pltpu.make_async_copy(
            oout_v.at[p],
            out_hbm.at[pl.ds(r0 - 2 * rchunk, rchunk)],
            sout.at[p]).wait()

      ob = oout_v.at[p]
      xb = xin_v.at[p]
      for j in range(0, J // L, 2):
        iva = idx_v[pl.ds(j * L, L)]
        ivb = idx_v[pl.ds((j + 1) * L, L)]

        @plsc.parallel_loop(0, rchunk, RU)
        def _rows(r):
          vals = [plsc.load_gather(xb.at[r + kk], [iva])
                  for kk in range(RU)]
          valsb = [plsc.load_gather(xb.at[r + kk], [ivb])
                   for kk in range(RU)]
          for kk in range(RU):
            ob[r + kk, pl.ds(j * L, L)] = vals[kk]
            ob[r + kk, pl.ds((j + 1) * L, L)] = valsb[kk]

      pltpu.async_copy(ob, out_hbm.at[pl.ds(r0, rchunk)], sout.at[p])

    lax.fori_loop(0, nchunks, chunk_body, None)

    end0 = base + (nchunks - 2) * rchunk
    pltpu.make_async_copy(oout_v.at[0], out_hbm.at[pl.ds(end0, rchunk)],
                          sout.at[0]).wait()
    pltpu.make_async_copy(oout_v.at[1],
                          out_hbm.at[pl.ds(end0 + rchunk, rchunk)],
                          sout.at[1]).wait()

  return k


def kernel(x, hierarchy_mapping_idx):
  B, C = x.shape
  J = hierarchy_mapping_idx.shape[0]
  rows_per_w = B // 32
  rchunk = 32
  k = _make_sc_kernel(B, C, J, rows_per_w, rchunk)
  return k(x, hierarchy_mapping_idx.astype(jnp.int32))


# speedup vs baseline: 1.4927x; 1.0114x over previous
import functools

import jax
import jax.numpy as jnp
from jax import lax
from jax.experimental import pallas as pl
from jax.experimental.pallas import tpu as pltpu
from jax.experimental.pallas import tpu_sc as plsc

L = 16
RU = 8


def _make_sc_kernel(B, C, J, rows_per_w, rchunk):
  nchunks = rows_per_w // rchunk
  xsz = rchunk * C
  mesh = plsc.VectorSubcoreMesh(core_axis_name="c", subcore_axis_name="s")

  @functools.partial(
      pl.kernel,
      mesh=mesh,
      out_type=jax.ShapeDtypeStruct((B, J), jnp.float32),
      scratch_types=[
          pltpu.VMEM((J,), jnp.int32),
          pltpu.VMEM((2, rchunk, C), jnp.float32),
          pltpu.VMEM((2, rchunk, J), jnp.float32),
          pltpu.SemaphoreType.DMA((2,)),
          pltpu.SemaphoreType.DMA((2,)),
      ],
      compiler_params=pltpu.CompilerParams(needs_layout_passes=False),
  )
  def k(x_hbm, idx_hbm, out_hbm, idx_v, xin_v, oout_v, sin, sout):
    wid = lax.axis_index("s") * 2 + lax.axis_index("c")
    base = wid * rows_per_w

    pltpu.sync_copy(idx_hbm, idx_v)
    pltpu.async_copy(x_hbm.at[pl.ds(base, rchunk)], xin_v.at[0], sin.at[0])

    def chunk_body(ci, _):
      p = lax.rem(ci, 2)
      r0 = base + ci * rchunk

      @pl.when(ci + 1 < nchunks)
      def _prefetch():
        pltpu.async_copy(x_hbm.at[pl.ds(r0 + rchunk, rchunk)],
                         xin_v.at[1 - p], sin.at[1 - p])

      pltpu.make_async_copy(x_hbm.at[pl.ds(r0, rchunk)],
                            xin_v.at[p], sin.at[p]).wait()

      @pl.when(ci >= 2)
      def _drain():
        pltpu.make_async_copy(
            oout_v.at[p],
            out_hbm.at[pl.ds(r0 - 2 * rchunk, rchunk)],
            sout.at[p]).wait()

      ob = oout_v.at[p]
      xb = xin_v.at[p]
      for j in range(0, J // L, 4):
        ivs = [idx_v[pl.ds((j + u) * L, L)] for u in range(4)]

        @plsc.parallel_loop(0, rchunk, RU)
        def _rows(r):
          vals = [[plsc.load_gather(xb.at[r + kk], [ivs[u]])
                   for kk in range(RU)] for u in range(4)]
          for u in range(4):
            for kk in range(RU):
              ob[r + kk, pl.ds((j + u) * L, L)] = vals[u][kk]

      pltpu.async_copy(ob, out_hbm.at[pl.ds(r0, rchunk)], sout.at[p])

    lax.fori_loop(0, nchunks, chunk_body, None)

    end0 = base + (nchunks - 2) * rchunk
    pltpu.make_async_copy(oout_v.at[0], out_hbm.at[pl.ds(end0, rchunk)],
                          sout.at[0]).wait()
    pltpu.make_async_copy(oout_v.at[1],
                          out_hbm.at[pl.ds(end0 + rchunk, rchunk)],
                          sout.at[1]).wait()

  return k


def kernel(x, hierarchy_mapping_idx):
  B, C = x.shape
  J = hierarchy_mapping_idx.shape[0]
  rows_per_w = B // 32
  rchunk = 32
  k = _make_sc_kernel(B, C, J, rows_per_w, rchunk)
  return k(x, hierarchy_mapping_idx.astype(jnp.int32))
